# baseline (speedup 1.0000x reference)
import functools

import jax
import jax.numpy as jnp
from jax import lax
from jax.experimental import pallas as pl
from jax.experimental.pallas import tpu as pltpu
from jax.experimental.pallas import tpu_sc as plsc

D = 128
CH = 128
R = 5


def _gather_body(n_ch, per_w, nc, table_hbm, idx_hbm, out_hbm,
                 idx_v, *scratch):
    bufs = scratch[:R]
    sem_g = scratch[R:2 * R]
    sem_w = scratch[2 * R:3 * R]
    wid = lax.axis_index("s") * nc + lax.axis_index("c")
    base = wid * per_w
    pltpu.sync_copy(idx_hbm.at[wid], idx_v)

    def start_gather(g, b):
        pltpu.async_copy(table_hbm.at[idx_v.at[g]], bufs[b], sem_g[b])

    def wait_gather(b):
        pltpu.make_async_copy(table_hbm.at[idx_v.at[0]], bufs[b],
                              sem_g[b]).wait()

    def start_write(g, b):
        pltpu.async_copy(bufs[b], out_hbm.at[pl.ds(base + g * CH, CH)],
                         sem_w[b])

    def wait_write(b):
        pltpu.make_async_copy(bufs[b], out_hbm.at[pl.ds(base, CH)],
                              sem_w[b]).wait()

    for b in range(R):
        start_gather(b, b)

    def group(i, _):
        g0 = R * i
        for b in range(R):
            wait_gather(b)
            start_write(g0 + b, b)

        @pl.when(g0 + R < n_ch)
        def _():
            for b in range(R):
                wait_write(b)
                start_gather(g0 + R + b, b)

        return 0

    lax.fori_loop(0, n_ch // R, group, 0)
    for b in range(R):
        wait_write(b)


def kernel(table, x):
    B, H = x.shape
    N = B * H
    info = plsc.get_sparse_core_info()
    nc, ns = info.num_cores, info.num_subcores
    nw = nc * ns
    per_w = N // nw
    n_ch = per_w // CH
    idx = x.reshape(nw, n_ch, CH).astype(jnp.int32)

    mesh = plsc.VectorSubcoreMesh(core_axis_name="c", subcore_axis_name="s")
    body = functools.partial(_gather_body, n_ch, per_w, nc)
    out = pl.kernel(
        body,
        mesh=mesh,
        out_type=jax.ShapeDtypeStruct((N, D), jnp.float32),
        scratch_types=(
            [pltpu.VMEM((n_ch, CH), jnp.int32)]
            + [pltpu.VMEM((CH, D), jnp.float32) for _ in range(R)]
            + [pltpu.SemaphoreType.DMA for _ in range(2 * R)]
        ),
    )(table, idx)
    return out.reshape(B, H, D)


# speedup vs baseline: 1.7846x; 1.7846x over previous
import functools

import jax
import jax.numpy as jnp
from jax import lax
from jax.experimental import pallas as pl
from jax.experimental.pallas import tpu as pltpu
from jax.experimental.pallas import tpu_sc as plsc

D = 128
BB = 2
R = 4


def _gather_body(n_ch, b_per_w, H, nc, table_hbm, idx_hbm, out_hbm,
                 idx_v, *scratch):
    bufs = scratch[:R]
    sem_g = scratch[R:2 * R]
    sem_w = scratch[2 * R:3 * R]
    wid = lax.axis_index("s") * nc + lax.axis_index("c")
    base = wid * b_per_w
    pltpu.sync_copy(idx_hbm.at[wid], idx_v)

    def start_gather(g, b):
        pltpu.async_copy(table_hbm.at[idx_v.at[g]], bufs[b], sem_g[b])

    def wait_gather(b):
        pltpu.make_async_copy(table_hbm.at[idx_v.at[0]], bufs[b],
                              sem_g[b]).wait()

    def start_write(g, b):
        for j in range(BB):
            pltpu.async_copy(bufs[b].at[pl.ds(j * H, H)],
                             out_hbm.at[base + g * BB + j], sem_w[b])

    def wait_write(b):
        for j in range(BB):
            pltpu.make_async_copy(bufs[b].at[pl.ds(j * H, H)],
                                  out_hbm.at[base], sem_w[b]).wait()

    for b in range(R):
        start_gather(b, b)

    def group(i, _):
        g0 = R * i
        for b in range(R):
            wait_gather(b)
            start_write(g0 + b, b)

        @pl.when(g0 + R < n_ch)
        def _():
            for b in range(R):
                wait_write(b)
                start_gather(g0 + R + b, b)

        return 0

    lax.fori_loop(0, n_ch // R, group, 0)
    for b in range(R):
        wait_write(b)


def kernel(table, x):
    B, H = x.shape
    N = B * H
    info = plsc.get_sparse_core_info()
    nc, ns = info.num_cores, info.num_subcores
    nw = nc * ns
    b_per_w = B // nw
    n_ch = b_per_w // BB
    idx = x.reshape(nw, n_ch, BB * H).astype(jnp.int32)

    mesh = plsc.VectorSubcoreMesh(core_axis_name="c", subcore_axis_name="s")
    body = functools.partial(_gather_body, n_ch, b_per_w, H, nc)
    out = pl.kernel(
        body,
        mesh=mesh,
        out_type=jax.ShapeDtypeStruct((B, H, D), jnp.float32),
        scratch_types=(
            [pltpu.VMEM((n_ch, BB * H), jnp.int32)]
            + [pltpu.VMEM((BB * H, D), jnp.float32) for _ in range(R)]
            + [pltpu.SemaphoreType.DMA for _ in range(2 * R)]
        ),
    )(table, idx)
    return out


# speedup vs baseline: 3.0697x; 1.7201x over previous
import functools

import jax
import jax.numpy as jnp
from jax import lax
from jax.experimental import pallas as pl
from jax.experimental.pallas import tpu as pltpu
from jax.experimental.pallas import tpu_sc as plsc

D = 128
CH = 128
R = 5


def _gather_body(n_ch, per_w, nc, table_hbm, idx_hbm, out_hbm,
                 idx_v, *scratch):
    bufs = scratch[:R]
    sem_g = scratch[R:2 * R]
    sem_w = scratch[2 * R:3 * R]
    wid = lax.axis_index("s") * nc + lax.axis_index("c")
    base = wid * per_w
    pltpu.sync_copy(idx_hbm.at[wid], idx_v)

    def start_gather(g, b):
        pltpu.async_copy(table_hbm.at[idx_v.at[g]], bufs[b], sem_g[b])

    def wait_gather(b):
        pltpu.make_async_copy(table_hbm.at[idx_v.at[0]], bufs[b],
                              sem_g[b]).wait()

    def start_write(g, b):
        pltpu.async_copy(bufs[b], out_hbm.at[pl.ds(base + g * CH, CH)],
                         sem_w[b])

    def wait_write(b):
        pltpu.make_async_copy(bufs[b], out_hbm.at[pl.ds(base, CH)],
                              sem_w[b]).wait()

    for b in range(R):
        start_gather(b, b)

    def group(i, _):
        g0 = R * i
        for b in range(R):
            wait_gather(b)
            start_write(g0 + b, b)

        @pl.when(g0 + R < n_ch)
        def _():
            for b in range(R):
                wait_write(b)
                start_gather(g0 + R + b, b)

        return 0

    lax.fori_loop(0, n_ch // R, group, 0)
    for b in range(R):
        wait_write(b)


def kernel(table, x):
    B, H = x.shape
    N = B * H
    info = plsc.get_sparse_core_info()
    nc, ns = info.num_cores, info.num_subcores
    nw = nc * ns
    per_w = N // nw
    n_ch = per_w // CH
    idx = x.T.reshape(nw, n_ch, CH).astype(jnp.int32)

    mesh = plsc.VectorSubcoreMesh(core_axis_name="c", subcore_axis_name="s")
    body = functools.partial(_gather_body, n_ch, per_w, nc)
    out = pl.kernel(
        body,
        mesh=mesh,
        out_type=jax.ShapeDtypeStruct((N, D), jnp.float32),
        scratch_types=(
            [pltpu.VMEM((n_ch, CH), jnp.int32)]
            + [pltpu.VMEM((CH, D), jnp.float32) for _ in range(R)]
            + [pltpu.SemaphoreType.DMA for _ in range(2 * R)]
        ),
    )(table, idx)
    return out.reshape(H, B, D).transpose(1, 0, 2)
